# baseline (speedup 1.0000x reference)
import functools

import jax
import jax.numpy as jnp
from jax import lax
from jax.experimental import pallas as pl
from jax.experimental.pallas import tpu as pltpu
from jax.experimental.pallas import tpu_sc as plsc

N = 10000
E = 320000
F_IN, F_H, F_OUT = 128, 32, 40

NC, NS = 1, 16
NW = NC * NS
SUB = 128
CROWS = 8
EROWS = 2560
EP = EROWS * SUB
ROWS_PER_W = EROWS // NW
NCHUNK = ROWS_PER_W // CROWS
NPAD = 10240
RPT = NPAD // NS

_mesh = plsc.VectorSubcoreMesh(core_axis_name="c", subcore_axis_name="s",
                               num_cores=NC)


def _agg_body(g_hbm, src_hbm, dst_hbm, zero_hbm, out_hbm,
              src_v, dst_v, rows_v, acc, gsem, ssem):
    c = lax.axis_index("c")
    s = lax.axis_index("s")
    wid = s * NC + c
    zbase = s * RPT
    pltpu.sync_copy(zero_hbm.at[pl.ds(zbase, RPT)], acc.at[pl.ds(zbase, RPT)])
    plsc.subcore_barrier()

    erow0 = wid * ROWS_PER_W

    def chunk(t, carry):
        r0 = erow0 + t * CROWS
        pltpu.sync_copy(src_hbm.at[pl.ds(r0, CROWS)], src_v)
        pltpu.sync_copy(dst_hbm.at[pl.ds(r0, CROWS)], dst_v)
        gathers = [
            pltpu.async_copy(g_hbm.at[src_v.at[j]], rows_v.at[j], gsem)
            for j in range(CROWS)
        ]
        for h in gathers:
            h.wait()
        scats = [
            pltpu.async_copy(rows_v.at[j], acc.at[dst_v.at[j]], ssem, add=True)
            for j in range(CROWS)
        ]
        for h in scats:
            h.wait()
        return carry

    lax.fori_loop(0, NCHUNK, chunk, 0)
    plsc.subcore_barrier()
    pltpu.sync_copy(acc.at[pl.ds(zbase, RPT)], out_hbm.at[c, pl.ds(zbase, RPT)])


_agg = functools.partial(
    pl.kernel,
    out_type=jax.ShapeDtypeStruct((NC, NPAD, F_H), jnp.float32),
    mesh=_mesh,
    compiler_params=pltpu.CompilerParams(use_tc_tiling_on_sc=False),
    scratch_types=[
        pltpu.VMEM((CROWS, SUB), jnp.int32),
        pltpu.VMEM((CROWS, SUB), jnp.int32),
        pltpu.VMEM((CROWS, SUB, F_H), jnp.float32),
        pltpu.VMEM_SHARED((NPAD, F_H), jnp.float32),
        pltpu.SemaphoreType.DMA,
        pltpu.SemaphoreType.DMA,
    ],
)(_agg_body)


def _deg_body(dst_hbm, ones_hbm, zero_hbm, out_hbm,
              dst_v, ones_v, acc, ssem):
    c = lax.axis_index("c")
    s = lax.axis_index("s")
    wid = s * NC + c
    zbase = s * RPT
    pltpu.sync_copy(zero_hbm.at[pl.ds(zbase, RPT)], acc.at[pl.ds(zbase, RPT)])
    pltpu.sync_copy(ones_hbm, ones_v)
    plsc.subcore_barrier()

    erow0 = wid * ROWS_PER_W

    def chunk(t, carry):
        r0 = erow0 + t * CROWS
        pltpu.sync_copy(dst_hbm.at[pl.ds(r0, CROWS)], dst_v)
        scats = [
            pltpu.async_copy(ones_v, acc.at[dst_v.at[j]], ssem, add=True)
            for j in range(CROWS)
        ]
        for h in scats:
            h.wait()
        return carry

    lax.fori_loop(0, NCHUNK, chunk, 0)
    plsc.subcore_barrier()
    pltpu.sync_copy(acc.at[pl.ds(zbase, RPT)], out_hbm.at[c, pl.ds(zbase, RPT)])


_deg = functools.partial(
    pl.kernel,
    out_type=jax.ShapeDtypeStruct((NC, NPAD, 16), jnp.float32),
    mesh=_mesh,
    compiler_params=pltpu.CompilerParams(use_tc_tiling_on_sc=False),
    scratch_types=[
        pltpu.VMEM((CROWS, SUB), jnp.int32),
        pltpu.VMEM((SUB, 16), jnp.float32),
        pltpu.VMEM_SHARED((NPAD, 16), jnp.float32),
        pltpu.SemaphoreType.DMA,
    ],
)(_deg_body)


_BLK = 2000
_GRID = N // _BLK


def _tc1_body(x_ref, w1_ref, degp_ref, g1_ref, dinv_ref):
    h = jnp.dot(x_ref[...], w1_ref[...], preferred_element_type=jnp.float32)
    deg = jnp.sum(degp_ref[:, :, 0:1], axis=0) + 1.0
    dinv = lax.rsqrt(deg)
    g1_ref[...] = h * dinv
    dinv_ref[...] = dinv


def _tc2_body(p_ref, g1_ref, dinv_ref, b1_ref, g2_ref):
    ssum = jnp.sum(p_ref[...], axis=0) + g1_ref[...]
    h = jnp.maximum(dinv_ref[...] * ssum + b1_ref[...], 0.0)
    g2_ref[...] = h * dinv_ref[...]


def _tc3_body(p_ref, g2_ref, dinv_ref, w2_ref, b2_ref, o_ref):
    t = dinv_ref[...] * (jnp.sum(p_ref[...], axis=0) + g2_ref[...])
    logits = jnp.dot(t, w2_ref[...], preferred_element_type=jnp.float32)
    logits = logits + b2_ref[...]
    m = jnp.max(logits, axis=1, keepdims=True)
    ex = jnp.exp(logits - m)
    o_ref[...] = logits - m - jnp.log(jnp.sum(ex, axis=1, keepdims=True))


_tc1 = pl.pallas_call(
    _tc1_body,
    grid=(_GRID,),
    in_specs=[
        pl.BlockSpec((_BLK, F_IN), lambda i: (i, 0)),
        pl.BlockSpec((F_IN, F_H), lambda i: (0, 0)),
        pl.BlockSpec((NC, _BLK, 16), lambda i: (0, i, 0)),
    ],
    out_specs=[
        pl.BlockSpec((_BLK, F_H), lambda i: (i, 0)),
        pl.BlockSpec((_BLK, 1), lambda i: (i, 0)),
    ],
    out_shape=[
        jax.ShapeDtypeStruct((N, F_H), jnp.float32),
        jax.ShapeDtypeStruct((N, 1), jnp.float32),
    ],
)

_tc2 = pl.pallas_call(
    _tc2_body,
    grid=(_GRID,),
    in_specs=[
        pl.BlockSpec((NC, _BLK, F_H), lambda i: (0, i, 0)),
        pl.BlockSpec((_BLK, F_H), lambda i: (i, 0)),
        pl.BlockSpec((_BLK, 1), lambda i: (i, 0)),
        pl.BlockSpec((1, F_H), lambda i: (0, 0)),
    ],
    out_specs=pl.BlockSpec((_BLK, F_H), lambda i: (i, 0)),
    out_shape=jax.ShapeDtypeStruct((N, F_H), jnp.float32),
)

_tc3 = pl.pallas_call(
    _tc3_body,
    grid=(_GRID,),
    in_specs=[
        pl.BlockSpec((NC, _BLK, F_H), lambda i: (0, i, 0)),
        pl.BlockSpec((_BLK, F_H), lambda i: (i, 0)),
        pl.BlockSpec((_BLK, 1), lambda i: (i, 0)),
        pl.BlockSpec((F_H, F_OUT), lambda i: (0, 0)),
        pl.BlockSpec((1, F_OUT), lambda i: (0, 0)),
    ],
    out_specs=pl.BlockSpec((_BLK, F_OUT), lambda i: (i, 0)),
    out_shape=jax.ShapeDtypeStruct((N, F_OUT), jnp.float32),
)


def kernel(x, edge_index, W1, b1, W2, b2):
    src = edge_index[0].astype(jnp.int32)
    dst = edge_index[1].astype(jnp.int32)
    pad = EP - E
    src_p = jnp.concatenate([src, jnp.zeros((pad,), jnp.int32)]).reshape(EROWS, SUB)
    dst_p = jnp.concatenate([dst, jnp.full((pad,), N, jnp.int32)]).reshape(EROWS, SUB)
    z16 = jnp.zeros((NPAD, 16), jnp.float32)
    z32 = jnp.zeros((NPAD, F_H), jnp.float32)
    ones_rows = jnp.zeros((SUB, 16), jnp.float32).at[:, 0].set(1.0)

    degp = _deg(dst_p, ones_rows, z16)
    g1, dinv = _tc1(x, W1, degp[:, :N, :])
    a1 = _agg(g1, src_p, dst_p, z32)
    g2 = _tc2(a1[:, :N, :], g1, dinv, b1.reshape(1, F_H))
    a2 = _agg(g2, src_p, dst_p, z32)
    return _tc3(a2[:, :N, :], g2, dinv, W2, b2.reshape(1, F_OUT))


# speedup vs baseline: 1.1119x; 1.1119x over previous
import functools

import jax
import jax.numpy as jnp
from jax import lax
from jax.experimental import pallas as pl
from jax.experimental.pallas import tpu as pltpu
from jax.experimental.pallas import tpu_sc as plsc

N = 10000
E = 320000
F_IN, F_H, F_OUT = 128, 32, 40

NC, NS = 2, 16
NW = NC * NS
SUB = 128
CROWS = 8
EROWS = 2560
EP = EROWS * SUB
ROWS_PER_W = EROWS // NW
NCHUNK = ROWS_PER_W // CROWS
NPAD = 10240
RPT = NPAD // NS

_mesh = plsc.VectorSubcoreMesh(core_axis_name="c", subcore_axis_name="s",
                               num_cores=NC)


def _agg_body(g_hbm, e_hbm, zero_hbm, out_hbm,
              idx_v, rows_v, acc, gsem, isem, ssem0, ssem1):
    c = lax.axis_index("c")
    s = lax.axis_index("s")
    wid = s * NC + c
    zbase = s * RPT
    ssems = (ssem0, ssem1)
    zcp = pltpu.async_copy(zero_hbm.at[pl.ds(zbase, RPT)],
                           acc.at[pl.ds(zbase, RPT)], gsem)

    erow0 = wid * ROWS_PER_W

    def idx_load(t):
        r0 = erow0 + t * CROWS
        return pltpu.async_copy(e_hbm.at[pl.ds(r0, CROWS)],
                                idx_v.at[t % 3], isem)

    icp = idx_load(0)
    zcp.wait()
    plsc.subcore_barrier()

    scats = {}
    for t in range(NCHUNK):
        rb = t % 2
        if t - 2 >= 0:
            for h in scats.pop(t - 2):
                h.wait()
        icp.wait()
        gathers = [
            pltpu.async_copy(g_hbm.at[idx_v.at[t % 3, j, 0]],
                             rows_v.at[rb, j], gsem)
            for j in range(CROWS)
        ]
        if t + 1 < NCHUNK:
            icp = idx_load(t + 1)
        for h in gathers:
            h.wait()
        scats[t] = [
            pltpu.async_copy(rows_v.at[rb, j],
                             acc.at[idx_v.at[t % 3, j, 1]],
                             ssems[rb], add=True)
            for j in range(CROWS)
        ]
    for t in sorted(scats):
        for h in scats[t]:
            h.wait()
    plsc.subcore_barrier()
    pltpu.sync_copy(acc.at[pl.ds(zbase, RPT)], out_hbm.at[c, pl.ds(zbase, RPT)])


_agg = functools.partial(
    pl.kernel,
    out_type=jax.ShapeDtypeStruct((NC, NPAD, F_H), jnp.float32),
    mesh=_mesh,
    compiler_params=pltpu.CompilerParams(use_tc_tiling_on_sc=False),
    scratch_types=[
        pltpu.VMEM((3, CROWS, 2, SUB), jnp.int32),
        pltpu.VMEM((2, CROWS, SUB, F_H), jnp.float32),
        pltpu.VMEM_SHARED((NPAD, F_H), jnp.float32),
        pltpu.SemaphoreType.DMA,
        pltpu.SemaphoreType.DMA,
        pltpu.SemaphoreType.DMA,
        pltpu.SemaphoreType.DMA,
    ],
)(_agg_body)


def _deg_body(dst_hbm, ones_hbm, zero_hbm, out_hbm,
              dst_v, ones_v, acc, isem, ssem0, ssem1):
    c = lax.axis_index("c")
    s = lax.axis_index("s")
    wid = s * NC + c
    zbase = s * RPT
    ssems = (ssem0, ssem1)
    zcp = pltpu.async_copy(zero_hbm.at[pl.ds(zbase, RPT)],
                           acc.at[pl.ds(zbase, RPT)], ssem0)
    pltpu.sync_copy(ones_hbm, ones_v)

    erow0 = wid * ROWS_PER_W

    def idx_load(t):
        r0 = erow0 + t * CROWS
        return pltpu.async_copy(dst_hbm.at[pl.ds(r0, CROWS)],
                                dst_v.at[t % 3], isem)

    icp = idx_load(0)
    zcp.wait()
    plsc.subcore_barrier()

    scats = {}
    for t in range(NCHUNK):
        if t - 2 >= 0:
            for h in scats.pop(t - 2):
                h.wait()
        icp.wait()
        if t + 1 < NCHUNK:
            icp2 = idx_load(t + 1)
        scats[t] = [
            pltpu.async_copy(ones_v, acc.at[dst_v.at[t % 3, j]],
                             ssems[t % 2], add=True)
            for j in range(CROWS)
        ]
        if t + 1 < NCHUNK:
            icp = icp2
    for t in sorted(scats):
        for h in scats[t]:
            h.wait()
    plsc.subcore_barrier()
    pltpu.sync_copy(acc.at[pl.ds(zbase, RPT)], out_hbm.at[c, pl.ds(zbase, RPT)])


_deg = functools.partial(
    pl.kernel,
    out_type=jax.ShapeDtypeStruct((NC, NPAD, 16), jnp.float32),
    mesh=_mesh,
    compiler_params=pltpu.CompilerParams(use_tc_tiling_on_sc=False),
    scratch_types=[
        pltpu.VMEM((3, CROWS, SUB), jnp.int32),
        pltpu.VMEM((SUB, 16), jnp.float32),
        pltpu.VMEM_SHARED((NPAD, 16), jnp.float32),
        pltpu.SemaphoreType.DMA,
        pltpu.SemaphoreType.DMA,
        pltpu.SemaphoreType.DMA,
    ],
)(_deg_body)


_BLK = 2000
_GRID = N // _BLK


def _tc1_body(x_ref, w1_ref, degp_ref, g1_ref, dinv_ref):
    h = jnp.dot(x_ref[...], w1_ref[...], preferred_element_type=jnp.float32)
    deg = jnp.sum(degp_ref[:, :, 0:1], axis=0) + 1.0
    dinv = lax.rsqrt(deg)
    g1_ref[...] = h * dinv
    dinv_ref[...] = dinv


def _tc2_body(p_ref, g1_ref, dinv_ref, b1_ref, g2_ref):
    ssum = jnp.sum(p_ref[...], axis=0) + g1_ref[...]
    h = jnp.maximum(dinv_ref[...] * ssum + b1_ref[...], 0.0)
    g2_ref[...] = h * dinv_ref[...]


def _tc3_body(p_ref, g2_ref, dinv_ref, w2_ref, b2_ref, o_ref):
    t = dinv_ref[...] * (jnp.sum(p_ref[...], axis=0) + g2_ref[...])
    logits = jnp.dot(t, w2_ref[...], preferred_element_type=jnp.float32)
    logits = logits + b2_ref[...]
    m = jnp.max(logits, axis=1, keepdims=True)
    ex = jnp.exp(logits - m)
    o_ref[...] = logits - m - jnp.log(jnp.sum(ex, axis=1, keepdims=True))


_tc1 = pl.pallas_call(
    _tc1_body,
    grid=(_GRID,),
    in_specs=[
        pl.BlockSpec((_BLK, F_IN), lambda i: (i, 0)),
        pl.BlockSpec((F_IN, F_H), lambda i: (0, 0)),
        pl.BlockSpec((NC, _BLK, 16), lambda i: (0, i, 0)),
    ],
    out_specs=[
        pl.BlockSpec((_BLK, F_H), lambda i: (i, 0)),
        pl.BlockSpec((_BLK, 1), lambda i: (i, 0)),
    ],
    out_shape=[
        jax.ShapeDtypeStruct((N, F_H), jnp.float32),
        jax.ShapeDtypeStruct((N, 1), jnp.float32),
    ],
)

_tc2 = pl.pallas_call(
    _tc2_body,
    grid=(_GRID,),
    in_specs=[
        pl.BlockSpec((NC, _BLK, F_H), lambda i: (0, i, 0)),
        pl.BlockSpec((_BLK, F_H), lambda i: (i, 0)),
        pl.BlockSpec((_BLK, 1), lambda i: (i, 0)),
        pl.BlockSpec((1, F_H), lambda i: (0, 0)),
    ],
    out_specs=pl.BlockSpec((_BLK, F_H), lambda i: (i, 0)),
    out_shape=jax.ShapeDtypeStruct((N, F_H), jnp.float32),
)

_tc3 = pl.pallas_call(
    _tc3_body,
    grid=(_GRID,),
    in_specs=[
        pl.BlockSpec((NC, _BLK, F_H), lambda i: (0, i, 0)),
        pl.BlockSpec((_BLK, F_H), lambda i: (i, 0)),
        pl.BlockSpec((_BLK, 1), lambda i: (i, 0)),
        pl.BlockSpec((F_H, F_OUT), lambda i: (0, 0)),
        pl.BlockSpec((1, F_OUT), lambda i: (0, 0)),
    ],
    out_specs=pl.BlockSpec((_BLK, F_OUT), lambda i: (i, 0)),
    out_shape=jax.ShapeDtypeStruct((N, F_OUT), jnp.float32),
)


def kernel(x, edge_index, W1, b1, W2, b2):
    src = edge_index[0].astype(jnp.int32)
    dst = edge_index[1].astype(jnp.int32)
    pad = EP - E
    src_p = jnp.concatenate([src, jnp.zeros((pad,), jnp.int32)]).reshape(EROWS, SUB)
    dst_p = jnp.concatenate([dst, jnp.full((pad,), N, jnp.int32)]).reshape(EROWS, SUB)
    e_p = jnp.stack([src_p, dst_p], axis=1)
    z16 = jnp.zeros((NPAD, 16), jnp.float32)
    z32 = jnp.zeros((NPAD, F_H), jnp.float32)
    ones_rows = jnp.zeros((SUB, 16), jnp.float32).at[:, 0].set(1.0)

    degp = _deg(dst_p, ones_rows, z16)
    g1, dinv = _tc1(x, W1, degp[:, :N, :])
    a1 = _agg(g1, e_p, z32)
    g2 = _tc2(a1[:, :N, :], g1, dinv, b1.reshape(1, F_H))
    a2 = _agg(g2, e_p, z32)
    return _tc3(a2[:, :N, :], g2, dinv, W2, b2.reshape(1, F_OUT))


# speedup vs baseline: 1.2290x; 1.1052x over previous
import functools

import jax
import jax.numpy as jnp
from jax import lax
from jax.experimental import pallas as pl
from jax.experimental.pallas import tpu as pltpu
from jax.experimental.pallas import tpu_sc as plsc

N = 10000
E = 320000
F_IN, F_H, F_OUT = 128, 32, 40

NC, NS = 2, 16
NW = NC * NS
SUB = 128
CROWS = 8
EROWS = 2560
EP = EROWS * SUB
ROWS_PER_W = EROWS // NW
NCHUNK = ROWS_PER_W // CROWS
NPAD = 10240
RPT = NPAD // NS

_mesh = plsc.VectorSubcoreMesh(core_axis_name="c", subcore_axis_name="s",
                               num_cores=NC)


def _agg_body(g_hbm, e_hbm, zero_hbm, out_hbm,
              idx_v, rows_v, acc, gsem, isem, ssem0, ssem1):
    c = lax.axis_index("c")
    s = lax.axis_index("s")
    wid = s * NC + c
    zbase = s * RPT
    ssems = (ssem0, ssem1)
    zcp = pltpu.async_copy(zero_hbm.at[pl.ds(zbase, RPT)],
                           acc.at[pl.ds(zbase, RPT)], gsem)

    def idx_load(t):
        return pltpu.async_copy(e_hbm.at[wid, t], idx_v.at[t % 3], isem)

    icp = idx_load(0)
    zcp.wait()
    plsc.subcore_barrier()

    scats = {}
    for t in range(NCHUNK):
        rb = t % 2
        if t - 2 >= 0:
            scats.pop(t - 2).wait()
        icp.wait()
        gcp = pltpu.async_copy(g_hbm.at[idx_v.at[t % 3, 0]],
                               rows_v.at[rb], gsem)
        if t + 1 < NCHUNK:
            icp = idx_load(t + 1)
        gcp.wait()
        scats[t] = pltpu.async_copy(rows_v.at[rb],
                                    acc.at[idx_v.at[t % 3, 1]],
                                    ssems[rb], add=True)
    for t in sorted(scats):
        scats[t].wait()
    plsc.subcore_barrier()
    pltpu.sync_copy(acc.at[pl.ds(zbase, RPT)], out_hbm.at[c, pl.ds(zbase, RPT)])


_agg = functools.partial(
    pl.kernel,
    out_type=jax.ShapeDtypeStruct((NC, NPAD, F_H), jnp.float32),
    mesh=_mesh,
    compiler_params=pltpu.CompilerParams(use_tc_tiling_on_sc=False),
    scratch_types=[
        pltpu.VMEM((3, 2, CROWS * SUB), jnp.int32),
        pltpu.VMEM((2, CROWS * SUB, F_H), jnp.float32),
        pltpu.VMEM_SHARED((NPAD, F_H), jnp.float32),
        pltpu.SemaphoreType.DMA,
        pltpu.SemaphoreType.DMA,
        pltpu.SemaphoreType.DMA,
        pltpu.SemaphoreType.DMA,
    ],
)(_agg_body)


def _deg_body(e_hbm, ones_hbm, zero_hbm, out_hbm,
              dst_v, ones_v, acc, isem, ssem0, ssem1):
    c = lax.axis_index("c")
    s = lax.axis_index("s")
    wid = s * NC + c
    zbase = s * RPT
    ssems = (ssem0, ssem1)
    zcp = pltpu.async_copy(zero_hbm.at[pl.ds(zbase, RPT)],
                           acc.at[pl.ds(zbase, RPT)], ssem0)
    pltpu.sync_copy(ones_hbm, ones_v)

    def idx_load(t):
        return pltpu.async_copy(e_hbm.at[wid, t, 1], dst_v.at[t % 3], isem)

    icp = idx_load(0)
    zcp.wait()
    plsc.subcore_barrier()

    scats = {}
    for t in range(NCHUNK):
        if t - 2 >= 0:
            scats.pop(t - 2).wait()
        icp.wait()
        if t + 1 < NCHUNK:
            icp2 = idx_load(t + 1)
        scats[t] = pltpu.async_copy(ones_v, acc.at[dst_v.at[t % 3]],
                                    ssems[t % 2], add=True)
        if t + 1 < NCHUNK:
            icp = icp2
    for t in sorted(scats):
        scats[t].wait()
    plsc.subcore_barrier()
    pltpu.sync_copy(acc.at[pl.ds(zbase, RPT)], out_hbm.at[c, pl.ds(zbase, RPT)])


_deg = functools.partial(
    pl.kernel,
    out_type=jax.ShapeDtypeStruct((NC, NPAD, 16), jnp.float32),
    mesh=_mesh,
    compiler_params=pltpu.CompilerParams(use_tc_tiling_on_sc=False),
    scratch_types=[
        pltpu.VMEM((3, CROWS * SUB), jnp.int32),
        pltpu.VMEM((CROWS * SUB, 16), jnp.float32),
        pltpu.VMEM_SHARED((NPAD, 16), jnp.float32),
        pltpu.SemaphoreType.DMA,
        pltpu.SemaphoreType.DMA,
        pltpu.SemaphoreType.DMA,
    ],
)(_deg_body)


_BLK = 2000
_GRID = N // _BLK


def _tc1_body(x_ref, w1_ref, degp_ref, g1_ref, dinv_ref):
    h = jnp.dot(x_ref[...], w1_ref[...], preferred_element_type=jnp.float32)
    deg = jnp.sum(degp_ref[:, :, 0:1], axis=0) + 1.0
    dinv = lax.rsqrt(deg)
    g1_ref[...] = h * dinv
    dinv_ref[...] = dinv


def _tc2_body(p_ref, g1_ref, dinv_ref, b1_ref, g2_ref):
    ssum = jnp.sum(p_ref[...], axis=0) + g1_ref[...]
    h = jnp.maximum(dinv_ref[...] * ssum + b1_ref[...], 0.0)
    g2_ref[...] = h * dinv_ref[...]


def _tc3_body(p_ref, g2_ref, dinv_ref, w2_ref, b2_ref, o_ref):
    t = dinv_ref[...] * (jnp.sum(p_ref[...], axis=0) + g2_ref[...])
    logits = jnp.dot(t, w2_ref[...], preferred_element_type=jnp.float32)
    logits = logits + b2_ref[...]
    m = jnp.max(logits, axis=1, keepdims=True)
    ex = jnp.exp(logits - m)
    o_ref[...] = logits - m - jnp.log(jnp.sum(ex, axis=1, keepdims=True))


_tc1 = pl.pallas_call(
    _tc1_body,
    grid=(_GRID,),
    in_specs=[
        pl.BlockSpec((_BLK, F_IN), lambda i: (i, 0)),
        pl.BlockSpec((F_IN, F_H), lambda i: (0, 0)),
        pl.BlockSpec((NC, _BLK, 16), lambda i: (0, i, 0)),
    ],
    out_specs=[
        pl.BlockSpec((_BLK, F_H), lambda i: (i, 0)),
        pl.BlockSpec((_BLK, 1), lambda i: (i, 0)),
    ],
    out_shape=[
        jax.ShapeDtypeStruct((N, F_H), jnp.float32),
        jax.ShapeDtypeStruct((N, 1), jnp.float32),
    ],
)

_tc2 = pl.pallas_call(
    _tc2_body,
    grid=(_GRID,),
    in_specs=[
        pl.BlockSpec((NC, _BLK, F_H), lambda i: (0, i, 0)),
        pl.BlockSpec((_BLK, F_H), lambda i: (i, 0)),
        pl.BlockSpec((_BLK, 1), lambda i: (i, 0)),
        pl.BlockSpec((1, F_H), lambda i: (0, 0)),
    ],
    out_specs=pl.BlockSpec((_BLK, F_H), lambda i: (i, 0)),
    out_shape=jax.ShapeDtypeStruct((N, F_H), jnp.float32),
)

_tc3 = pl.pallas_call(
    _tc3_body,
    grid=(_GRID,),
    in_specs=[
        pl.BlockSpec((NC, _BLK, F_H), lambda i: (0, i, 0)),
        pl.BlockSpec((_BLK, F_H), lambda i: (i, 0)),
        pl.BlockSpec((_BLK, 1), lambda i: (i, 0)),
        pl.BlockSpec((F_H, F_OUT), lambda i: (0, 0)),
        pl.BlockSpec((1, F_OUT), lambda i: (0, 0)),
    ],
    out_specs=pl.BlockSpec((_BLK, F_OUT), lambda i: (i, 0)),
    out_shape=jax.ShapeDtypeStruct((N, F_OUT), jnp.float32),
)


def kernel(x, edge_index, W1, b1, W2, b2):
    src = edge_index[0].astype(jnp.int32)
    dst = edge_index[1].astype(jnp.int32)
    pad = EP - E
    src_p = jnp.concatenate([src, jnp.zeros((pad,), jnp.int32)])
    dst_p = jnp.concatenate([dst, jnp.full((pad,), N, jnp.int32)])
    src_r = src_p.reshape(NW, NCHUNK, CROWS * SUB)
    dst_r = dst_p.reshape(NW, NCHUNK, CROWS * SUB)
    e_p = jnp.stack([src_r, dst_r], axis=2)
    z16 = jnp.zeros((NPAD, 16), jnp.float32)
    z32 = jnp.zeros((NPAD, F_H), jnp.float32)
    ones_rows = jnp.zeros((CROWS * SUB, 16), jnp.float32).at[:, 0].set(1.0)

    degp = _deg(e_p, ones_rows, z16)
    g1, dinv = _tc1(x, W1, degp[:, :N, :])
    a1 = _agg(g1, e_p, z32)
    g2 = _tc2(a1[:, :N, :], g1, dinv, b1.reshape(1, F_H))
    a2 = _agg(g2, e_p, z32)
    return _tc3(a2[:, :N, :], g2, dinv, W2, b2.reshape(1, F_OUT))


# speedup vs baseline: 2.1515x; 1.7507x over previous
import functools

import jax
import jax.numpy as jnp
from jax import lax
from jax.experimental import pallas as pl
from jax.experimental.pallas import tpu as pltpu
from jax.experimental.pallas import tpu_sc as plsc

N = 10000
E = 320000
F_IN, F_H, F_OUT = 128, 32, 40

NC, NS = 2, 16
NW = NC * NS
SUB = 128
CROWS = 8
EROWS = 2560
EP = EROWS * SUB
ROWS_PER_W = EROWS // NW
NCHUNK = ROWS_PER_W // CROWS
NPAD = 10240
RPT = NPAD // NS

_mesh = plsc.VectorSubcoreMesh(core_axis_name="c", subcore_axis_name="s",
                               num_cores=NC)


def _agg_body(g_hbm, e_hbm, zero_hbm, out_hbm,
              idx_v, rows_v, acc, gtab, gsem, isem, ssem0, ssem1):
    c = lax.axis_index("c")
    s = lax.axis_index("s")
    wid = s * NC + c
    zbase = s * RPT
    gbase = s * (N // NS)
    ssems = (ssem0, ssem1)
    zcp = pltpu.async_copy(zero_hbm.at[pl.ds(zbase, RPT)],
                           acc.at[pl.ds(zbase, RPT)], gsem)
    tcp = pltpu.async_copy(g_hbm.at[pl.ds(gbase, N // NS)],
                           gtab.at[pl.ds(gbase, N // NS)], gsem)

    def idx_load(t):
        return pltpu.async_copy(e_hbm.at[wid, t], idx_v.at[t % 3], isem)

    icp = idx_load(0)
    zcp.wait()
    tcp.wait()
    plsc.subcore_barrier()

    scats = {}
    for t in range(NCHUNK):
        rb = t % 2
        if t - 2 >= 0:
            scats.pop(t - 2).wait()
        icp.wait()
        gcp = pltpu.async_copy(gtab.at[idx_v.at[t % 3, 0]],
                               rows_v.at[rb], gsem)
        if t + 1 < NCHUNK:
            icp = idx_load(t + 1)
        gcp.wait()
        scats[t] = pltpu.async_copy(rows_v.at[rb],
                                    acc.at[idx_v.at[t % 3, 1]],
                                    ssems[rb], add=True)
    for t in sorted(scats):
        scats[t].wait()
    plsc.subcore_barrier()
    pltpu.sync_copy(acc.at[pl.ds(zbase, RPT)], out_hbm.at[c, pl.ds(zbase, RPT)])


_agg = functools.partial(
    pl.kernel,
    out_type=jax.ShapeDtypeStruct((NC, NPAD, F_H), jnp.float32),
    mesh=_mesh,
    compiler_params=pltpu.CompilerParams(use_tc_tiling_on_sc=False),
    scratch_types=[
        pltpu.VMEM((3, 2, CROWS * SUB), jnp.int32),
        pltpu.VMEM((2, CROWS * SUB, F_H), jnp.float32),
        pltpu.VMEM_SHARED((NPAD, F_H), jnp.float32),
        pltpu.VMEM_SHARED((N, F_H), jnp.float32),
        pltpu.SemaphoreType.DMA,
        pltpu.SemaphoreType.DMA,
        pltpu.SemaphoreType.DMA,
        pltpu.SemaphoreType.DMA,
    ],
)(_agg_body)


def _deg_body(e_hbm, ones_hbm, zero_hbm, out_hbm,
              dst_v, ones_v, acc, isem, ssem0, ssem1):
    c = lax.axis_index("c")
    s = lax.axis_index("s")
    wid = s * NC + c
    zbase = s * RPT
    ssems = (ssem0, ssem1)
    zcp = pltpu.async_copy(zero_hbm.at[pl.ds(zbase, RPT)],
                           acc.at[pl.ds(zbase, RPT)], ssem0)
    pltpu.sync_copy(ones_hbm, ones_v)

    def idx_load(t):
        return pltpu.async_copy(e_hbm.at[wid, t, 1], dst_v.at[t % 3], isem)

    icp = idx_load(0)
    zcp.wait()
    plsc.subcore_barrier()

    scats = {}
    for t in range(NCHUNK):
        if t - 2 >= 0:
            scats.pop(t - 2).wait()
        icp.wait()
        if t + 1 < NCHUNK:
            icp2 = idx_load(t + 1)
        scats[t] = pltpu.async_copy(ones_v, acc.at[dst_v.at[t % 3]],
                                    ssems[t % 2], add=True)
        if t + 1 < NCHUNK:
            icp = icp2
    for t in sorted(scats):
        scats[t].wait()
    plsc.subcore_barrier()
    pltpu.sync_copy(acc.at[pl.ds(zbase, RPT)], out_hbm.at[c, pl.ds(zbase, RPT)])


_deg = functools.partial(
    pl.kernel,
    out_type=jax.ShapeDtypeStruct((NC, NPAD, 16), jnp.float32),
    mesh=_mesh,
    compiler_params=pltpu.CompilerParams(use_tc_tiling_on_sc=False),
    scratch_types=[
        pltpu.VMEM((3, CROWS * SUB), jnp.int32),
        pltpu.VMEM((CROWS * SUB, 16), jnp.float32),
        pltpu.VMEM_SHARED((NPAD, 16), jnp.float32),
        pltpu.SemaphoreType.DMA,
        pltpu.SemaphoreType.DMA,
        pltpu.SemaphoreType.DMA,
    ],
)(_deg_body)


_BLK = 2000
_GRID = N // _BLK


def _tc1_body(x_ref, w1_ref, degp_ref, g1_ref, dinv_ref):
    h = jnp.dot(x_ref[...], w1_ref[...], preferred_element_type=jnp.float32)
    deg = jnp.sum(degp_ref[:, :, 0:1], axis=0) + 1.0
    dinv = lax.rsqrt(deg)
    g1_ref[...] = h * dinv
    dinv_ref[...] = dinv


def _tc2_body(p_ref, g1_ref, dinv_ref, b1_ref, g2_ref):
    ssum = jnp.sum(p_ref[...], axis=0) + g1_ref[...]
    h = jnp.maximum(dinv_ref[...] * ssum + b1_ref[...], 0.0)
    g2_ref[...] = h * dinv_ref[...]


def _tc3_body(p_ref, g2_ref, dinv_ref, w2_ref, b2_ref, o_ref):
    t = dinv_ref[...] * (jnp.sum(p_ref[...], axis=0) + g2_ref[...])
    logits = jnp.dot(t, w2_ref[...], preferred_element_type=jnp.float32)
    logits = logits + b2_ref[...]
    m = jnp.max(logits, axis=1, keepdims=True)
    ex = jnp.exp(logits - m)
    o_ref[...] = logits - m - jnp.log(jnp.sum(ex, axis=1, keepdims=True))


_tc1 = pl.pallas_call(
    _tc1_body,
    grid=(_GRID,),
    in_specs=[
        pl.BlockSpec((_BLK, F_IN), lambda i: (i, 0)),
        pl.BlockSpec((F_IN, F_H), lambda i: (0, 0)),
        pl.BlockSpec((NC, _BLK, 16), lambda i: (0, i, 0)),
    ],
    out_specs=[
        pl.BlockSpec((_BLK, F_H), lambda i: (i, 0)),
        pl.BlockSpec((_BLK, 1), lambda i: (i, 0)),
    ],
    out_shape=[
        jax.ShapeDtypeStruct((N, F_H), jnp.float32),
        jax.ShapeDtypeStruct((N, 1), jnp.float32),
    ],
)

_tc2 = pl.pallas_call(
    _tc2_body,
    grid=(_GRID,),
    in_specs=[
        pl.BlockSpec((NC, _BLK, F_H), lambda i: (0, i, 0)),
        pl.BlockSpec((_BLK, F_H), lambda i: (i, 0)),
        pl.BlockSpec((_BLK, 1), lambda i: (i, 0)),
        pl.BlockSpec((1, F_H), lambda i: (0, 0)),
    ],
    out_specs=pl.BlockSpec((_BLK, F_H), lambda i: (i, 0)),
    out_shape=jax.ShapeDtypeStruct((N, F_H), jnp.float32),
)

_tc3 = pl.pallas_call(
    _tc3_body,
    grid=(_GRID,),
    in_specs=[
        pl.BlockSpec((NC, _BLK, F_H), lambda i: (0, i, 0)),
        pl.BlockSpec((_BLK, F_H), lambda i: (i, 0)),
        pl.BlockSpec((_BLK, 1), lambda i: (i, 0)),
        pl.BlockSpec((F_H, F_OUT), lambda i: (0, 0)),
        pl.BlockSpec((1, F_OUT), lambda i: (0, 0)),
    ],
    out_specs=pl.BlockSpec((_BLK, F_OUT), lambda i: (i, 0)),
    out_shape=jax.ShapeDtypeStruct((N, F_OUT), jnp.float32),
)


def kernel(x, edge_index, W1, b1, W2, b2):
    src = edge_index[0].astype(jnp.int32)
    dst = edge_index[1].astype(jnp.int32)
    pad = EP - E
    src_p = jnp.concatenate([src, jnp.zeros((pad,), jnp.int32)])
    dst_p = jnp.concatenate([dst, jnp.full((pad,), N, jnp.int32)])
    src_r = src_p.reshape(NW, NCHUNK, CROWS * SUB)
    dst_r = dst_p.reshape(NW, NCHUNK, CROWS * SUB)
    e_p = jnp.stack([src_r, dst_r], axis=2)
    z16 = jnp.zeros((NPAD, 16), jnp.float32)
    z32 = jnp.zeros((NPAD, F_H), jnp.float32)
    ones_rows = jnp.zeros((CROWS * SUB, 16), jnp.float32).at[:, 0].set(1.0)

    degp = _deg(e_p, ones_rows, z16)
    g1, dinv = _tc1(x, W1, degp[:, :N, :])
    a1 = _agg(g1, e_p, z32)
    g2 = _tc2(a1[:, :N, :], g1, dinv, b1.reshape(1, F_H))
    a2 = _agg(g2, e_p, z32)
    return _tc3(a2[:, :N, :], g2, dinv, W2, b2.reshape(1, F_OUT))


# speedup vs baseline: 2.1810x; 1.0137x over previous
import functools

import jax
import jax.numpy as jnp
from jax import lax
from jax.experimental import pallas as pl
from jax.experimental.pallas import tpu as pltpu
from jax.experimental.pallas import tpu_sc as plsc

N = 10000
E = 320000
F_IN, F_H, F_OUT = 128, 32, 40

NC, NS = 2, 16
NW = NC * NS
SUB = 128
CROWS = 8
EROWS = 2560
EP = EROWS * SUB
ROWS_PER_W = EROWS // NW
NCHUNK = ROWS_PER_W // CROWS
NPAD = 10016
RPT = NPAD // NS

_mesh = plsc.VectorSubcoreMesh(core_axis_name="c", subcore_axis_name="s",
                               num_cores=NC)


def _agg_body(g_hbm, e_hbm, zero_hbm, out_hbm,
              idx_v, rows_v, acc, gtab, gsem, isem, ssem0, ssem1):
    c = lax.axis_index("c")
    s = lax.axis_index("s")
    wid = s * NC + c
    zbase = s * RPT
    gbase = s * (N // NS)
    ssems = (ssem0, ssem1)
    zcp = pltpu.async_copy(zero_hbm.at[pl.ds(zbase, RPT)],
                           acc.at[pl.ds(zbase, RPT)], gsem)
    tcp = pltpu.async_copy(g_hbm.at[pl.ds(gbase, N // NS)],
                           gtab.at[pl.ds(gbase, N // NS)], gsem)

    def idx_load(t):
        return pltpu.async_copy(e_hbm.at[wid, t], idx_v.at[t % 3], isem)

    icp = idx_load(0)
    zcp.wait()
    tcp.wait()
    plsc.subcore_barrier()

    scats = {}
    for t in range(NCHUNK):
        rb = t % 2
        if t - 2 >= 0:
            scats.pop(t - 2).wait()
        icp.wait()
        gcp = pltpu.async_copy(gtab.at[idx_v.at[t % 3, 0]],
                               rows_v.at[rb], gsem)
        if t + 1 < NCHUNK:
            icp = idx_load(t + 1)
        gcp.wait()
        scats[t] = pltpu.async_copy(rows_v.at[rb],
                                    acc.at[idx_v.at[t % 3, 1]],
                                    ssems[rb], add=True)
    for t in sorted(scats):
        scats[t].wait()
    plsc.subcore_barrier()
    pltpu.sync_copy(acc.at[pl.ds(zbase, RPT)], out_hbm.at[c, pl.ds(zbase, RPT)])


_agg = functools.partial(
    pl.kernel,
    out_type=jax.ShapeDtypeStruct((NC, NPAD, F_H), jnp.float32),
    mesh=_mesh,
    compiler_params=pltpu.CompilerParams(use_tc_tiling_on_sc=False),
    scratch_types=[
        pltpu.VMEM((3, 2, CROWS * SUB), jnp.int32),
        pltpu.VMEM((2, CROWS * SUB, F_H), jnp.float32),
        pltpu.VMEM_SHARED((NPAD, F_H), jnp.float32),
        pltpu.VMEM_SHARED((N, F_H), jnp.float32),
        pltpu.SemaphoreType.DMA,
        pltpu.SemaphoreType.DMA,
        pltpu.SemaphoreType.DMA,
        pltpu.SemaphoreType.DMA,
    ],
)(_agg_body)


def _deg_body(e_hbm, ones_hbm, zero_hbm, out_hbm,
              dst_v, ones_v, acc, isem, ssem0, ssem1):
    c = lax.axis_index("c")
    s = lax.axis_index("s")
    wid = s * NC + c
    zbase = s * RPT
    ssems = (ssem0, ssem1)
    zcp = pltpu.async_copy(zero_hbm.at[pl.ds(zbase, RPT)],
                           acc.at[pl.ds(zbase, RPT)], ssem0)
    pltpu.sync_copy(ones_hbm, ones_v)

    def idx_load(t):
        return pltpu.async_copy(e_hbm.at[wid, t, 1], dst_v.at[t % 3], isem)

    icp = idx_load(0)
    zcp.wait()
    plsc.subcore_barrier()

    scats = {}
    for t in range(NCHUNK):
        if t - 2 >= 0:
            scats.pop(t - 2).wait()
        icp.wait()
        if t + 1 < NCHUNK:
            icp2 = idx_load(t + 1)
        scats[t] = pltpu.async_copy(ones_v, acc.at[dst_v.at[t % 3]],
                                    ssems[t % 2], add=True)
        if t + 1 < NCHUNK:
            icp = icp2
    for t in sorted(scats):
        scats[t].wait()
    plsc.subcore_barrier()
    pltpu.sync_copy(acc.at[pl.ds(zbase, RPT)], out_hbm.at[c, pl.ds(zbase, RPT)])


_deg = functools.partial(
    pl.kernel,
    out_type=jax.ShapeDtypeStruct((NC, NPAD, 8), jnp.float32),
    mesh=_mesh,
    compiler_params=pltpu.CompilerParams(use_tc_tiling_on_sc=False),
    scratch_types=[
        pltpu.VMEM((3, CROWS * SUB), jnp.int32),
        pltpu.VMEM((CROWS * SUB, 8), jnp.float32),
        pltpu.VMEM_SHARED((NPAD, 8), jnp.float32),
        pltpu.SemaphoreType.DMA,
        pltpu.SemaphoreType.DMA,
        pltpu.SemaphoreType.DMA,
    ],
)(_deg_body)


_BLK = 2000
_GRID = N // _BLK


def _tc1_body(x_ref, w1_ref, degp_ref, g1_ref, dinv_ref):
    h = jnp.dot(x_ref[...], w1_ref[...], preferred_element_type=jnp.float32)
    deg = jnp.sum(degp_ref[:, :, 0:1], axis=0) + 1.0
    dinv = lax.rsqrt(deg)
    g1_ref[...] = h * dinv
    dinv_ref[...] = dinv


def _tc2_body(p_ref, g1_ref, dinv_ref, b1_ref, g2_ref):
    ssum = jnp.sum(p_ref[...], axis=0) + g1_ref[...]
    h = jnp.maximum(dinv_ref[...] * ssum + b1_ref[...], 0.0)
    g2_ref[...] = h * dinv_ref[...]


def _tc3_body(p_ref, g2_ref, dinv_ref, w2_ref, b2_ref, o_ref):
    t = dinv_ref[...] * (jnp.sum(p_ref[...], axis=0) + g2_ref[...])
    logits = jnp.dot(t, w2_ref[...], preferred_element_type=jnp.float32)
    logits = logits + b2_ref[...]
    m = jnp.max(logits, axis=1, keepdims=True)
    ex = jnp.exp(logits - m)
    o_ref[...] = logits - m - jnp.log(jnp.sum(ex, axis=1, keepdims=True))


_tc1 = pl.pallas_call(
    _tc1_body,
    grid=(_GRID,),
    in_specs=[
        pl.BlockSpec((_BLK, F_IN), lambda i: (i, 0)),
        pl.BlockSpec((F_IN, F_H), lambda i: (0, 0)),
        pl.BlockSpec((NC, _BLK, 8), lambda i: (0, i, 0)),
    ],
    out_specs=[
        pl.BlockSpec((_BLK, F_H), lambda i: (i, 0)),
        pl.BlockSpec((_BLK, 1), lambda i: (i, 0)),
    ],
    out_shape=[
        jax.ShapeDtypeStruct((N, F_H), jnp.float32),
        jax.ShapeDtypeStruct((N, 1), jnp.float32),
    ],
)

_tc2 = pl.pallas_call(
    _tc2_body,
    grid=(_GRID,),
    in_specs=[
        pl.BlockSpec((NC, _BLK, F_H), lambda i: (0, i, 0)),
        pl.BlockSpec((_BLK, F_H), lambda i: (i, 0)),
        pl.BlockSpec((_BLK, 1), lambda i: (i, 0)),
        pl.BlockSpec((1, F_H), lambda i: (0, 0)),
    ],
    out_specs=pl.BlockSpec((_BLK, F_H), lambda i: (i, 0)),
    out_shape=jax.ShapeDtypeStruct((N, F_H), jnp.float32),
)

_tc3 = pl.pallas_call(
    _tc3_body,
    grid=(_GRID,),
    in_specs=[
        pl.BlockSpec((NC, _BLK, F_H), lambda i: (0, i, 0)),
        pl.BlockSpec((_BLK, F_H), lambda i: (i, 0)),
        pl.BlockSpec((_BLK, 1), lambda i: (i, 0)),
        pl.BlockSpec((F_H, F_OUT), lambda i: (0, 0)),
        pl.BlockSpec((1, F_OUT), lambda i: (0, 0)),
    ],
    out_specs=pl.BlockSpec((_BLK, F_OUT), lambda i: (i, 0)),
    out_shape=jax.ShapeDtypeStruct((N, F_OUT), jnp.float32),
)


def kernel(x, edge_index, W1, b1, W2, b2):
    src = edge_index[0].astype(jnp.int32)
    dst = edge_index[1].astype(jnp.int32)
    pad = EP - E
    src_p = jnp.concatenate([src, jnp.zeros((pad,), jnp.int32)])
    dst_p = jnp.concatenate([dst, jnp.full((pad,), N, jnp.int32)])
    src_r = src_p.reshape(NW, NCHUNK, CROWS * SUB)
    dst_r = dst_p.reshape(NW, NCHUNK, CROWS * SUB)
    e_p = jnp.stack([src_r, dst_r], axis=2)
    z16 = jnp.zeros((NPAD, 8), jnp.float32)
    z32 = jnp.zeros((NPAD, F_H), jnp.float32)
    ones_rows = jnp.zeros((CROWS * SUB, 8), jnp.float32).at[:, 0].set(1.0)

    degp = _deg(e_p, ones_rows, z16)
    g1, dinv = _tc1(x, W1, degp[:, :N, :])
    a1 = _agg(g1, e_p, z32)
    g2 = _tc2(a1[:, :N, :], g1, dinv, b1.reshape(1, F_H))
    a2 = _agg(g2, e_p, z32)
    return _tc3(a2[:, :N, :], g2, dinv, W2, b2.reshape(1, F_OUT))
